# baseline (speedup 1.0000x reference)
import functools

import jax
import jax.numpy as jnp
import numpy as np
from jax import lax
from jax.experimental import pallas as pl
from jax.experimental.pallas import tpu as pltpu
from jax.experimental.pallas import tpu_sc as plsc

N_EDGES = 320000
N_TRI = 640000
EMB = 128
INT = 64
RBF_DIM = 6
SBF_DIM = 42

BLK_A = 1280
BLK_B = 2560
BLK_C = 1280


def _swish(v):
    return v * jax.nn.sigmoid(v)


def _dot(a, b):
    return jnp.dot(a, b, preferred_element_type=jnp.float32)


def _pre_body(x_ref, rbf_ref, wji_ref, bji_ref, wkj_ref, bkj_ref,
              wrbf_ref, wdown_ref, xji_ref, t_ref):
    xb = x_ref[:]
    x_ji = _swish(_dot(xb, wji_ref[:]) + bji_ref[:])
    x_kj = _swish(_dot(xb, wkj_ref[:]) + bkj_ref[:])
    rbf_p = _dot(rbf_ref[:], wrbf_ref[:])
    xji_ref[:] = x_ji
    t_ref[:] = _swish(_dot(x_kj * rbf_p, wdown_ref[:]))


def _sbf_body(sbf_ref, wsbf_ref, out_ref):
    out_ref[:] = _dot(sbf_ref[:], wsbf_ref[:])


def _post_body(acc0_ref, acc1_ref, xji_ref, x_ref,
               wup_ref, rb0w1_ref, rb0b1_ref, rb0w2_ref, rb0b2_ref,
               wfin_ref, bfin_ref,
               ra0w1_ref, ra0b1_ref, ra0w2_ref, ra0b2_ref,
               ra1w1_ref, ra1b1_ref, ra1w2_ref, ra1b2_ref,
               out_ref):
    x_kj = _swish(_dot(acc0_ref[:] + acc1_ref[:], wup_ref[:]))
    x2 = xji_ref[:] + x_kj
    h = _swish(_dot(x2, rb0w1_ref[:]) + rb0b1_ref[:])
    h = _swish(_dot(h, rb0w2_ref[:]) + rb0b2_ref[:])
    x2 = x2 + h
    x2 = _swish(_dot(x2, wfin_ref[:]) + bfin_ref[:])
    xo = x_ref[:] + x2
    h = _swish(_dot(xo, ra0w1_ref[:]) + ra0b1_ref[:])
    h = _swish(_dot(h, ra0w2_ref[:]) + ra0b2_ref[:])
    xo = xo + h
    h = _swish(_dot(xo, ra1w1_ref[:]) + ra1b1_ref[:])
    h = _swish(_dot(h, ra1w2_ref[:]) + ra1b2_ref[:])
    out_ref[:] = xo + h


def _row_spec(blk, width):
    return pl.BlockSpec((blk, width), lambda i: (i, 0))


def _full_spec(shape):
    return pl.BlockSpec(shape, lambda i: tuple(0 for _ in shape))


def _pre_stage(x, rbf, W_ji, b_ji, W_kj, b_kj, W_rbf, W_down):
    n = x.shape[0]
    grid = n // BLK_A
    return pl.pallas_call(
        _pre_body,
        grid=(grid,),
        in_specs=[
            _row_spec(BLK_A, EMB),
            _row_spec(BLK_A, RBF_DIM),
            _full_spec((EMB, EMB)),
            _full_spec((1, EMB)),
            _full_spec((EMB, EMB)),
            _full_spec((1, EMB)),
            _full_spec((RBF_DIM, EMB)),
            _full_spec((EMB, 128)),
        ],
        out_specs=[
            _row_spec(BLK_A, EMB),
            _row_spec(BLK_A, 128),
        ],
        out_shape=[
            jax.ShapeDtypeStruct((n, EMB), jnp.float32),
            jax.ShapeDtypeStruct((n, 128), jnp.float32),
        ],
    )(x, rbf, W_ji, b_ji.reshape(1, EMB), W_kj, b_kj.reshape(1, EMB),
      W_rbf, W_down)


def _sbf_stage(sbf, W_sbf):
    n = sbf.shape[0]
    grid = n // BLK_B
    return pl.pallas_call(
        _sbf_body,
        grid=(grid,),
        in_specs=[
            _row_spec(BLK_B, SBF_DIM),
            _full_spec((SBF_DIM, 128)),
        ],
        out_specs=_row_spec(BLK_B, 128),
        out_shape=jax.ShapeDtypeStruct((n, 128), jnp.float32),
    )(sbf, W_sbf)


def _post_stage(acc2, x_ji, x, W_up, rb0_W1, rb0_b1, rb0_W2, rb0_b2,
                W_final, b_final, ra0_W1, ra0_b1, ra0_W2, ra0_b2,
                ra1_W1, ra1_b1, ra1_W2, ra1_b2):
    n = x.shape[0]
    grid = n // BLK_C
    return pl.pallas_call(
        _post_body,
        grid=(grid,),
        in_specs=[
            _row_spec(BLK_C, INT),
            pl.BlockSpec((BLK_C, INT), lambda i: (i + N_EDGES // BLK_C, 0)),
            _row_spec(BLK_C, EMB),
            _row_spec(BLK_C, EMB),
            _full_spec((INT, EMB)),
            _full_spec((EMB, EMB)), _full_spec((1, EMB)),
            _full_spec((EMB, EMB)), _full_spec((1, EMB)),
            _full_spec((EMB, EMB)), _full_spec((1, EMB)),
            _full_spec((EMB, EMB)), _full_spec((1, EMB)),
            _full_spec((EMB, EMB)), _full_spec((1, EMB)),
            _full_spec((EMB, EMB)), _full_spec((1, EMB)),
            _full_spec((EMB, EMB)), _full_spec((1, EMB)),
        ],
        out_specs=_row_spec(BLK_C, EMB),
        out_shape=jax.ShapeDtypeStruct((n, EMB), jnp.float32),
    )(acc2, acc2, x_ji, x,
      W_up, rb0_W1, rb0_b1.reshape(1, EMB), rb0_W2, rb0_b2.reshape(1, EMB),
      W_final, b_final.reshape(1, EMB),
      ra0_W1, ra0_b1.reshape(1, EMB), ra0_W2, ra0_b2.reshape(1, EMB),
      ra1_W1, ra1_b1.reshape(1, EMB), ra1_W2, ra1_b2.reshape(1, EMB))


SC_CORES = 2
SC_TILES = 16
N_WORKERS = SC_CORES * SC_TILES
WID = 128
R_CHUNK = 4000
N_PASS = N_EDGES // R_CHUNK
ACC_ROWS = 4128
TRI_PER_TILE = N_TRI // N_WORKERS
BKT_SZ = TRI_PER_TILE + 128
SWIN = 4000
N_SWIN = TRI_PER_TILE // SWIN
BATCH = 128
ZROWS = 86
NCHUNK_V = N_PASS * 16
TIDM = (1 << 20) - 1


def _sc_triplet_body(t_hbm, sbfp_hbm, ide_hbm, idr_hbm, out_hbm,
                     acc, idrw, idew, pakbuf, idebuf,
                     cnts, bases, bases0,
                     tidst, idest, rel2d, row_a, row_b, p_buf, zbuf,
                     rampb, padpakb):
    cid = lax.axis_index("c")
    sid = lax.axis_index("s")
    wid = sid * SC_CORES + cid
    tri_base0 = wid * TRI_PER_TILE
    iota16 = lax.iota(jnp.int32, 16)
    zero16 = jnp.zeros((16,), jnp.float32)
    zero16i = jnp.zeros((16,), jnp.int32)

    def zinit(r, _):
        for c8 in range(INT // 16):
            zbuf[r, pl.ds(c8 * 16, 16)] = zero16
        return 0
    lax.fori_loop(0, ZROWS, zinit, 0)

    for k in range(BATCH // 16):
        rampb[pl.ds(k * 16, 16)] = (k * 16) + iota16
    for k in range(BATCH // 16):
        v = rampb[pl.ds(k * 16, 16)]
        padpakb[pl.ds(k * 16, 16)] = (v + R_CHUNK) << 20

    def cinit(j, _):
        cnts[pl.ds(j * 16, 16)] = zero16i
        return 0
    lax.fori_loop(0, NCHUNK_V // 16, cinit, 0)

    def count_win(w, _):
        pltpu.sync_copy(idr_hbm.at[pl.ds(tri_base0 + w * SWIN, SWIN)], idrw)

        def count_body(v, _):
            ids = idrw[pl.ds(v * 16, 16)]
            idx = (ids // R_CHUNK) * 16 + iota16
            cur = plsc.load_gather(cnts, [idx])
            plsc.store_scatter(cnts, [idx], cur + 1)
            return 0
        lax.fori_loop(0, SWIN // 16, count_body, 0)
        return 0
    lax.fori_loop(0, N_SWIN, count_win, 0)

    def bases_body(c, run):
        v = cnts[pl.ds(c * 16, 16)]
        s = plsc.cumsum(v)
        excl = (run + s) - v
        bases[pl.ds(c * 16, 16)] = excl
        bases0[pl.ds(c * 16, 16)] = excl
        return run + s[15]
    lax.fori_loop(0, N_PASS, bases_body, jnp.int32(0))
    bases0[pl.ds(NCHUNK_V, 16)] = jnp.broadcast_to(
        jnp.int32(TRI_PER_TILE), (16,))
    for k in range(128 // 16):
        pakbuf[pl.ds(TRI_PER_TILE + k * 16, 16)] = (
            ((R_CHUNK + k * 16) + iota16) << 20) | ((wid * 128 + k * 16)
                                                    + iota16)
        idebuf[pl.ds(TRI_PER_TILE + k * 16, 16)] = (wid * 128 + k * 16) + iota16

    def place_win(w, _):
        base = tri_base0 + w * SWIN
        pltpu.sync_copy(idr_hbm.at[pl.ds(base, SWIN)], idrw)
        pltpu.sync_copy(ide_hbm.at[pl.ds(base, SWIN)], idew)

        def place_body(v, _):
            ids = idrw[pl.ds(v * 16, 16)]
            chunk = ids // R_CHUNK
            idx = chunk * 16 + iota16
            pos = plsc.load_gather(bases, [idx])
            tid = (base + v * 16) + iota16
            rel = ids - chunk * R_CHUNK
            plsc.store_scatter(pakbuf, [pos], tid | (rel << 20))
            plsc.store_scatter(idebuf, [pos], idew[pl.ds(v * 16, 16)])
            plsc.store_scatter(bases, [idx], pos + 1)
            return 0
        lax.fori_loop(0, SWIN // 16, place_body, 0)
        return 0
    lax.fori_loop(0, N_SWIN, place_win, 0)

    def pass_body(p, _):
        lo = p * R_CHUNK
        for z in range(ACC_ROWS // SC_TILES // ZROWS):
            pltpu.sync_copy(
                zbuf, acc.at[pl.ds(sid * (ACC_ROWS // SC_TILES) + z * ZROWS,
                                   ZROWS)])
        plsc.subcore_barrier()
        start = bases0[pl.ds(p * 16, 16)][0]
        end = bases0[pl.ds(p * 16 + 16, 16)][0]
        nb = ((end - start) + BATCH - 1) >> 7

        def batch_body(b, _):
            off0 = start + b * BATCH
            for k in range(BATCH // 16):
                ok = rampb[pl.ds(k * 16, 16)] < (end - off0)
                pak = pakbuf[pl.ds(off0 + k * 16, 16)]
                tidst[pl.ds(k * 16, 16)] = pak & TIDM
                idest[pl.ds(k * 16, 16)] = idebuf[pl.ds(off0 + k * 16, 16)]
                padc = padpakb[pl.ds(k * 16, 16)]
                sel = jnp.where(ok, pak, padc)
                rel2d[0, pl.ds(k * 16, 16)] = lax.shift_right_logical(sel, 20)
            pltpu.sync_copy(t_hbm.at[idest], row_a)
            pltpu.sync_copy(sbfp_hbm.at[tidst], row_b)

            def mul_body(r, _):
                for c8 in range(INT // 16):
                    p_buf[r, pl.ds(c8 * 16, 16)] = (
                        row_a[r, pl.ds(c8 * 16, 16)]
                        * row_b[r, pl.ds(c8 * 16, 16)])
                return 0
            lax.fori_loop(0, BATCH, mul_body, 0)
            pltpu.sync_copy(p_buf, acc.at[rel2d.at[0]], add=True)
            return 0
        lax.fori_loop(0, nb, batch_body, 0)
        plsc.subcore_barrier()
        rows_out = 800

        @pl.when(sid < 5)
        def _():
            pltpu.sync_copy(
                acc.at[pl.ds(sid * rows_out, rows_out)],
                out_hbm.at[pl.ds(cid * N_EDGES + lo + sid * rows_out,
                                 rows_out)])
        plsc.subcore_barrier()
        return 0

    lax.fori_loop(0, N_PASS, pass_body, 0)


def _sc_triplet_stage(t, sbf_p, id_expand, id_reduce):
    mesh = plsc.VectorSubcoreMesh(core_axis_name="c", subcore_axis_name="s")
    k = pl.kernel(
        _sc_triplet_body,
        out_type=jax.ShapeDtypeStruct((SC_CORES * N_EDGES, INT), jnp.float32),
        mesh=mesh,
        scratch_types=[
            pltpu.VMEM_SHARED((ACC_ROWS, INT), jnp.float32),
            pltpu.VMEM((SWIN,), jnp.int32),
            pltpu.VMEM((SWIN,), jnp.int32),
            pltpu.VMEM((BKT_SZ,), jnp.int32),
            pltpu.VMEM((BKT_SZ,), jnp.int32),
            pltpu.VMEM((NCHUNK_V,), jnp.int32),
            pltpu.VMEM((NCHUNK_V,), jnp.int32),
            pltpu.VMEM((NCHUNK_V + 16,), jnp.int32),
            pltpu.VMEM((BATCH,), jnp.int32),
            pltpu.VMEM((BATCH,), jnp.int32),
            pltpu.VMEM((1, BATCH), jnp.int32),
            pltpu.VMEM((BATCH, WID), jnp.float32),
            pltpu.VMEM((BATCH, WID), jnp.float32),
            pltpu.VMEM((BATCH, INT), jnp.float32),
            pltpu.VMEM((ZROWS, INT), jnp.float32),
            pltpu.VMEM((BATCH,), jnp.int32),
            pltpu.VMEM((BATCH,), jnp.int32),
        ],
        compiler_params=pltpu.CompilerParams(needs_layout_passes=False),
    )
    return k(t, sbf_p, id_expand, id_reduce)


def kernel(x, rbf, sbf, id_expand_kj, id_reduce_ji, W_rbf1, W_rbf2, W_sbf1,
           W_sbf2, W_ji, b_ji, W_kj, b_kj, W_down, W_up, rb0_W1, rb0_b1,
           rb0_W2, rb0_b2, W_final, b_final, ra0_W1, ra0_b1, ra0_W2, ra0_b2,
           ra1_W1, ra1_b1, ra1_W2, ra1_b2):
    n_edges = x.shape[0]
    W_rbf = _dot(W_rbf1, W_rbf2)
    W_sbf = jnp.pad(_dot(W_sbf1, W_sbf2), ((0, 0), (0, 128 - INT)))
    W_down_p = jnp.pad(W_down, ((0, 0), (0, 128 - INT)))
    x_ji, t = _pre_stage(x, rbf, W_ji, b_ji, W_kj, b_kj, W_rbf, W_down_p)
    sbf_p = _sbf_stage(sbf, W_sbf)
    acc2 = _sc_triplet_stage(t, sbf_p, id_expand_kj, id_reduce_ji)
    return _post_stage(acc2, x_ji, x, W_up, rb0_W1, rb0_b1, rb0_W2, rb0_b2,
                       W_final, b_final, ra0_W1, ra0_b1, ra0_W2, ra0_b2,
                       ra1_W1, ra1_b1, ra1_W2, ra1_b2)


# speedup vs baseline: 2.3795x; 2.3795x over previous
import functools

import jax
import jax.numpy as jnp
import numpy as np
from jax import lax
from jax.experimental import pallas as pl
from jax.experimental.pallas import tpu as pltpu
from jax.experimental.pallas import tpu_sc as plsc

N_EDGES = 320000
N_TRI = 640000
EMB = 128
INT = 64
RBF_DIM = 6
SBF_DIM = 42

BLK_A = 1280
BLK_B = 2560
BLK_C = 1280


def _swish(v):
    return v * jax.nn.sigmoid(v)


def _dot(a, b):
    return jnp.dot(a, b, preferred_element_type=jnp.float32)


def _pre_body(x_ref, rbf_ref, wji_ref, bji_ref, wkj_ref, bkj_ref,
              wrbf_ref, wdown_ref, xji_ref, t_ref):
    xb = x_ref[:]
    x_ji = _swish(_dot(xb, wji_ref[:]) + bji_ref[:])
    x_kj = _swish(_dot(xb, wkj_ref[:]) + bkj_ref[:])
    rbf_p = _dot(rbf_ref[:], wrbf_ref[:])
    xji_ref[:] = x_ji
    t_ref[:] = _swish(_dot(x_kj * rbf_p, wdown_ref[:]))


def _sbf_body(sbf_ref, wsbf_ref, out_ref):
    out_ref[:] = _dot(sbf_ref[:], wsbf_ref[:])


def _post_body(acc_ref, xji_ref, x_ref,
               wup_ref, rb0w1_ref, rb0b1_ref, rb0w2_ref, rb0b2_ref,
               wfin_ref, bfin_ref,
               ra0w1_ref, ra0b1_ref, ra0w2_ref, ra0b2_ref,
               ra1w1_ref, ra1b1_ref, ra1w2_ref, ra1b2_ref,
               out_ref):
    x_kj = _swish(_dot(acc_ref[:], wup_ref[:]))
    x2 = xji_ref[:] + x_kj
    h = _swish(_dot(x2, rb0w1_ref[:]) + rb0b1_ref[:])
    h = _swish(_dot(h, rb0w2_ref[:]) + rb0b2_ref[:])
    x2 = x2 + h
    x2 = _swish(_dot(x2, wfin_ref[:]) + bfin_ref[:])
    xo = x_ref[:] + x2
    h = _swish(_dot(xo, ra0w1_ref[:]) + ra0b1_ref[:])
    h = _swish(_dot(h, ra0w2_ref[:]) + ra0b2_ref[:])
    xo = xo + h
    h = _swish(_dot(xo, ra1w1_ref[:]) + ra1b1_ref[:])
    h = _swish(_dot(h, ra1w2_ref[:]) + ra1b2_ref[:])
    out_ref[:] = xo + h


def _row_spec(blk, width):
    return pl.BlockSpec((blk, width), lambda i: (i, 0))


def _full_spec(shape):
    return pl.BlockSpec(shape, lambda i: tuple(0 for _ in shape))


def _pre_stage(x, rbf, W_ji, b_ji, W_kj, b_kj, W_rbf, W_down):
    n = x.shape[0]
    grid = n // BLK_A
    return pl.pallas_call(
        _pre_body,
        grid=(grid,),
        in_specs=[
            _row_spec(BLK_A, EMB),
            _row_spec(BLK_A, RBF_DIM),
            _full_spec((EMB, EMB)),
            _full_spec((1, EMB)),
            _full_spec((EMB, EMB)),
            _full_spec((1, EMB)),
            _full_spec((RBF_DIM, EMB)),
            _full_spec((EMB, 128)),
        ],
        out_specs=[
            _row_spec(BLK_A, EMB),
            _row_spec(BLK_A, 128),
        ],
        out_shape=[
            jax.ShapeDtypeStruct((n, EMB), jnp.float32),
            jax.ShapeDtypeStruct((n, 128), jnp.float32),
        ],
    )(x, rbf, W_ji, b_ji.reshape(1, EMB), W_kj, b_kj.reshape(1, EMB),
      W_rbf, W_down)


def _sbf_stage(sbf, W_sbf):
    n = sbf.shape[0]
    grid = n // BLK_B
    return pl.pallas_call(
        _sbf_body,
        grid=(grid,),
        in_specs=[
            _row_spec(BLK_B, SBF_DIM),
            _full_spec((SBF_DIM, 128)),
        ],
        out_specs=_row_spec(BLK_B, 128),
        out_shape=jax.ShapeDtypeStruct((n, 128), jnp.float32),
    )(sbf, W_sbf)


def _post_stage(acc, x_ji, x, W_up, rb0_W1, rb0_b1, rb0_W2, rb0_b2,
                W_final, b_final, ra0_W1, ra0_b1, ra0_W2, ra0_b2,
                ra1_W1, ra1_b1, ra1_W2, ra1_b2):
    n = x.shape[0]
    grid = n // BLK_C
    return pl.pallas_call(
        _post_body,
        grid=(grid,),
        in_specs=[
            _row_spec(BLK_C, INT),
            _row_spec(BLK_C, EMB),
            _row_spec(BLK_C, EMB),
            _full_spec((INT, EMB)),
            _full_spec((EMB, EMB)), _full_spec((1, EMB)),
            _full_spec((EMB, EMB)), _full_spec((1, EMB)),
            _full_spec((EMB, EMB)), _full_spec((1, EMB)),
            _full_spec((EMB, EMB)), _full_spec((1, EMB)),
            _full_spec((EMB, EMB)), _full_spec((1, EMB)),
            _full_spec((EMB, EMB)), _full_spec((1, EMB)),
            _full_spec((EMB, EMB)), _full_spec((1, EMB)),
        ],
        out_specs=_row_spec(BLK_C, EMB),
        out_shape=jax.ShapeDtypeStruct((n, EMB), jnp.float32),
    )(acc, x_ji, x,
      W_up, rb0_W1, rb0_b1.reshape(1, EMB), rb0_W2, rb0_b2.reshape(1, EMB),
      W_final, b_final.reshape(1, EMB),
      ra0_W1, ra0_b1.reshape(1, EMB), ra0_W2, ra0_b2.reshape(1, EMB),
      ra1_W1, ra1_b1.reshape(1, EMB), ra1_W2, ra1_b2.reshape(1, EMB))


SC_CORES = 2
SC_TILES = 16
WID = 128
R_CHUNK = 6400
N_PASS = N_EDGES // (SC_CORES * R_CHUNK)
ACC_ROWS = 6656
TRI_PER_TILE = N_TRI // SC_TILES
WIN = 8000
N_WIN = TRI_PER_TILE // WIN
SEL_SZ = WIN + 256
TRASH = WIN + 240
BATCH = 128
ZROWS = 104


def _sc_triplet_body(t_hbm, sbfp_hbm, ide_hbm, idr_hbm, out_hbm,
                     acc, idr_buf, ide_buf, relb, expb, tidb, rel2d,
                     row_a, row_b, p_buf, zbuf):
    cid = lax.axis_index("c")
    sid = lax.axis_index("s")
    tri_base0 = sid * TRI_PER_TILE
    iota16 = lax.iota(jnp.int32, 16)
    zero16 = jnp.zeros((16,), jnp.float32)

    def zinit(r, _):
        for c8 in range(INT // 16):
            zbuf[r, pl.ds(c8 * 16, 16)] = zero16
        return 0
    lax.fori_loop(0, ZROWS, zinit, 0)

    def binit(j, _):
        relb[pl.ds(j * 16, 16)] = R_CHUNK + iota16
        expb[pl.ds(j * 16, 16)] = iota16
        tidb[pl.ds(j * 16, 16)] = iota16
        return 0
    lax.fori_loop(0, SEL_SZ // 16, binit, 0)

    def pass_body(p, _):
        chunk = p * SC_CORES + cid
        lo = chunk * R_CHUNK
        for z in range(ACC_ROWS // SC_TILES // ZROWS):
            pltpu.sync_copy(
                zbuf, acc.at[pl.ds(sid * (ACC_ROWS // SC_TILES) + z * ZROWS,
                                   ZROWS)])
        plsc.subcore_barrier()

        def win_body(w, _):
            base = tri_base0 + w * WIN
            pltpu.sync_copy(idr_hbm.at[pl.ds(base, WIN)], idr_buf)
            pltpu.sync_copy(ide_hbm.at[pl.ds(base, WIN)], ide_buf)

            def scan_body(v, cnt):
                ids = idr_buf[pl.ds(v * 16, 16)]
                m = (ids >= lo) & (ids < lo + R_CHUNK)
                mi = jnp.where(m, 1, 0)
                s = plsc.cumsum(mi)
                pos = jnp.where(m, (cnt + s) - mi, TRASH + iota16)
                plsc.store_scatter(relb, [pos], ids - lo)
                plsc.store_scatter(expb, [pos], ide_buf[pl.ds(v * 16, 16)])
                tid = (base + v * 16) + iota16
                plsc.store_scatter(tidb, [pos], tid)
                return cnt + s[15]

            cnt = lax.fori_loop(0, WIN // 16, scan_body, jnp.int32(0))
            for k in range(BATCH // 16):
                relb[pl.ds(cnt + k * 16, 16)] = (R_CHUNK + k * 16) + iota16
                expb[pl.ds(cnt + k * 16, 16)] = (sid * 256 + k * 16) + iota16
                tidb[pl.ds(cnt + k * 16, 16)] = (sid * 256 + k * 16) + iota16
            nb = (cnt + BATCH - 1) >> 7

            def batch_body(b, _):
                off = b * BATCH
                pltpu.sync_copy(t_hbm.at[expb.at[pl.ds(off, BATCH)]], row_a)
                pltpu.sync_copy(sbfp_hbm.at[tidb.at[pl.ds(off, BATCH)]],
                                row_b)

                def mul_body(r, _):
                    for c8 in range(INT // 16):
                        p_buf[r, pl.ds(c8 * 16, 16)] = (
                            row_a[r, pl.ds(c8 * 16, 16)]
                            * row_b[r, pl.ds(c8 * 16, 16)])
                    return 0
                lax.fori_loop(0, BATCH, mul_body, 0)
                for k in range(BATCH // 16):
                    rel2d[0, pl.ds(k * 16, 16)] = relb[pl.ds(off + k * 16, 16)]
                pltpu.sync_copy(p_buf, acc.at[rel2d.at[0]], add=True)
                return 0

            lax.fori_loop(0, nb, batch_body, 0)
            return 0

        lax.fori_loop(0, N_WIN, win_body, 0)
        plsc.subcore_barrier()
        rows_out = 800

        @pl.when(sid < 8)
        def _():
            pltpu.sync_copy(acc.at[pl.ds(sid * rows_out, rows_out)],
                            out_hbm.at[pl.ds(lo + sid * rows_out, rows_out)])
        plsc.subcore_barrier()
        return 0

    lax.fori_loop(0, N_PASS, pass_body, 0)


def _sc_triplet_stage(t, sbf_p, id_expand, id_reduce):
    mesh = plsc.VectorSubcoreMesh(core_axis_name="c", subcore_axis_name="s")
    k = pl.kernel(
        _sc_triplet_body,
        out_type=jax.ShapeDtypeStruct((N_EDGES, INT), jnp.float32),
        mesh=mesh,
        scratch_types=[
            pltpu.VMEM_SHARED((ACC_ROWS, INT), jnp.float32),
            pltpu.VMEM((WIN,), jnp.int32),
            pltpu.VMEM((WIN,), jnp.int32),
            pltpu.VMEM((SEL_SZ,), jnp.int32),
            pltpu.VMEM((SEL_SZ,), jnp.int32),
            pltpu.VMEM((SEL_SZ,), jnp.int32),
            pltpu.VMEM((1, BATCH), jnp.int32),
            pltpu.VMEM((BATCH, WID), jnp.float32),
            pltpu.VMEM((BATCH, WID), jnp.float32),
            pltpu.VMEM((BATCH, INT), jnp.float32),
            pltpu.VMEM((ZROWS, INT), jnp.float32),
        ],
        compiler_params=pltpu.CompilerParams(needs_layout_passes=False),
    )
    return k(t, sbf_p, id_expand, id_reduce)


def kernel(x, rbf, sbf, id_expand_kj, id_reduce_ji, W_rbf1, W_rbf2, W_sbf1,
           W_sbf2, W_ji, b_ji, W_kj, b_kj, W_down, W_up, rb0_W1, rb0_b1,
           rb0_W2, rb0_b2, W_final, b_final, ra0_W1, ra0_b1, ra0_W2, ra0_b2,
           ra1_W1, ra1_b1, ra1_W2, ra1_b2):
    n_edges = x.shape[0]
    W_rbf = _dot(W_rbf1, W_rbf2)
    W_sbf = jnp.pad(_dot(W_sbf1, W_sbf2), ((0, 0), (0, 128 - INT)))
    W_down_p = jnp.pad(W_down, ((0, 0), (0, 128 - INT)))
    x_ji, t = _pre_stage(x, rbf, W_ji, b_ji, W_kj, b_kj, W_rbf, W_down_p)
    sbf_p = _sbf_stage(sbf, W_sbf)
    acc = _sc_triplet_stage(t, sbf_p, id_expand_kj, id_reduce_ji)
    return _post_stage(acc, x_ji, x, W_up, rb0_W1, rb0_b1, rb0_W2, rb0_b2,
                       W_final, b_final, ra0_W1, ra0_b1, ra0_W2, ra0_b2,
                       ra1_W1, ra1_b1, ra1_W2, ra1_b2)
